# Initial kernel scaffold; baseline (speedup 1.0000x reference)
import functools

import jax
import jax.numpy as jnp
from jax import lax
from jax.experimental import pallas as pl
from jax.experimental.pallas import tpu as pltpu
from jax.experimental.pallas import tpu_sc as plsc

NC = 2
NS = 16
NW = NC * NS
K = 80


def _sc_mesh():
    return plsc.VectorSubcoreMesh(core_axis_name="c", subcore_axis_name="s")


def _sc_degree(dst2d, ones, zeros, n_nodes):
    nb = dst2d.shape[0]
    rpw = nb // NW
    rps = n_nodes // NS

    @functools.partial(
        pl.kernel,
        out_type=jax.ShapeDtypeStruct((NC, n_nodes, 16), jnp.float32),
        mesh=_sc_mesh(),
        scratch_types=[
            pltpu.VMEM((rpw, K), jnp.int32),
            pltpu.VMEM((K, 16), jnp.float32),
            pltpu.VMEM_SHARED((n_nodes, 16), jnp.float32),
            pltpu.SemaphoreType.DMA,
        ],
    )
    def k(dst_hbm, ones_hbm, zeros_hbm, out_hbm, dst_v, ones_v, acc_sh, sem):
        c = lax.axis_index("c")
        s = lax.axis_index("s")
        w = c * NS + s
        pltpu.sync_copy(dst_hbm.at[pl.ds(w * rpw, rpw)], dst_v)
        pltpu.sync_copy(ones_hbm, ones_v)
        pltpu.sync_copy(zeros_hbm.at[pl.ds(s * rps, rps)],
                        acc_sh.at[pl.ds(s * rps, rps)])
        plsc.subcore_barrier()

        @pl.loop(0, rpw)
        def _(j):
            pltpu.sync_copy(ones_v, acc_sh.at[dst_v.at[j]], add=True)

        plsc.subcore_barrier()
        pltpu.sync_copy(acc_sh.at[pl.ds(s * rps, rps)],
                        out_hbm.at[c, pl.ds(s * rps, rps)])

    return k(dst2d, ones, zeros)


def _sc_propagate(hp, src2d, dst2d, zeros, n_nodes, d):
    nb = src2d.shape[0]
    rpw = nb // NW
    rps = n_nodes // NS

    @functools.partial(
        pl.kernel,
        out_type=jax.ShapeDtypeStruct((NC, n_nodes, d), jnp.float32),
        mesh=_sc_mesh(),
        scratch_types=[
            pltpu.VMEM((rpw, K), jnp.int32),
            pltpu.VMEM((rpw, K), jnp.int32),
            pltpu.VMEM((K, d), jnp.float32),
            pltpu.VMEM_SHARED((n_nodes, d), jnp.float32),
            pltpu.SemaphoreType.DMA,
        ],
    )
    def k(hp_hbm, src_hbm, dst_hbm, zeros_hbm, out_hbm,
          src_v, dst_v, rows_v, acc_sh, sem):
        c = lax.axis_index("c")
        s = lax.axis_index("s")
        w = c * NS + s
        pltpu.sync_copy(src_hbm.at[pl.ds(w * rpw, rpw)], src_v)
        pltpu.sync_copy(dst_hbm.at[pl.ds(w * rpw, rpw)], dst_v)
        pltpu.sync_copy(zeros_hbm.at[pl.ds(s * rps, rps)],
                        acc_sh.at[pl.ds(s * rps, rps)])
        plsc.subcore_barrier()

        @pl.loop(0, rpw)
        def _(j):
            pltpu.async_copy(hp_hbm.at[src_v.at[j]], rows_v, sem).wait()
            pltpu.sync_copy(rows_v, acc_sh.at[dst_v.at[j]], add=True)

        plsc.subcore_barrier()
        pltpu.sync_copy(acc_sh.at[pl.ds(s * rps, rps)],
                        out_hbm.at[c, pl.ds(s * rps, rps)])

    return k(hp, src2d, dst2d, zeros)


_BN = 1000


def _tc_matmul(x, w):
    n, kd = x.shape
    dout = w.shape[1]

    def body(x_ref, w_ref, o_ref):
        o_ref[...] = jnp.dot(x_ref[...], w_ref[...],
                             preferred_element_type=jnp.float32,
                             precision=lax.Precision.HIGHEST)

    return pl.pallas_call(
        body,
        grid=(n // _BN,),
        in_specs=[pl.BlockSpec((_BN, kd), lambda i: (i, 0)),
                  pl.BlockSpec((kd, dout), lambda i: (0, 0))],
        out_specs=pl.BlockSpec((_BN, dout), lambda i: (i, 0)),
        out_shape=jax.ShapeDtypeStruct((n, dout), jnp.float32),
    )(x, w)


def _tc_prep(deg0, deg1, h1):
    n, d = h1.shape

    def body(d0_ref, d1_ref, h_ref, dis_ref, hp_ref):
        deg = d0_ref[:, :1] + d1_ref[:, :1] + 1.0
        dis = lax.rsqrt(deg)
        dis_ref[...] = dis
        hp_ref[...] = dis * h_ref[...]

    return pl.pallas_call(
        body,
        grid=(n // _BN,),
        in_specs=[pl.BlockSpec((_BN, 16), lambda i: (i, 0)),
                  pl.BlockSpec((_BN, 16), lambda i: (i, 0)),
                  pl.BlockSpec((_BN, d), lambda i: (i, 0))],
        out_specs=[pl.BlockSpec((_BN, 1), lambda i: (i, 0)),
                   pl.BlockSpec((_BN, d), lambda i: (i, 0))],
        out_shape=[jax.ShapeDtypeStruct((n, 1), jnp.float32),
                   jax.ShapeDtypeStruct((n, d), jnp.float32)],
    )(deg0, deg1, h1)


def _tc_mid(acc_a, acc_b, dis, h1, b1, w2):
    n, d = h1.shape
    dout = w2.shape[1]

    def body(a_ref, b_ref, dis_ref, h1_ref, bias_ref, w2_ref, h2_ref, hp2_ref):
        dis = dis_ref[...]
        z = dis * (a_ref[...] + b_ref[...]) + (dis * dis) * h1_ref[...] \
            + bias_ref[...]
        h = jnp.maximum(z, 0.0)
        h2 = jnp.dot(h, w2_ref[...], preferred_element_type=jnp.float32,
                     precision=lax.Precision.HIGHEST)
        h2_ref[...] = h2
        hp2_ref[...] = dis * h2

    return pl.pallas_call(
        body,
        grid=(n // _BN,),
        in_specs=[pl.BlockSpec((_BN, d), lambda i: (i, 0)),
                  pl.BlockSpec((_BN, d), lambda i: (i, 0)),
                  pl.BlockSpec((_BN, 1), lambda i: (i, 0)),
                  pl.BlockSpec((_BN, d), lambda i: (i, 0)),
                  pl.BlockSpec((1, d), lambda i: (0, 0)),
                  pl.BlockSpec((d, dout), lambda i: (0, 0))],
        out_specs=[pl.BlockSpec((_BN, dout), lambda i: (i, 0)),
                   pl.BlockSpec((_BN, dout), lambda i: (i, 0))],
        out_shape=[jax.ShapeDtypeStruct((n, dout), jnp.float32),
                   jax.ShapeDtypeStruct((n, dout), jnp.float32)],
    )(acc_a, acc_b, dis, h1, b1, w2)


def _tc_final(acc_a, acc_b, dis, h2, b2):
    n, d = h2.shape

    def body(a_ref, b_ref, dis_ref, h2_ref, bias_ref, o_ref):
        dis = dis_ref[...]
        z = dis * (a_ref[...] + b_ref[...]) + (dis * dis) * h2_ref[...] \
            + bias_ref[...]
        m = jnp.max(z, axis=1, keepdims=True)
        lse = jnp.log(jnp.sum(jnp.exp(z - m), axis=1, keepdims=True)) + m
        o_ref[...] = z - lse

    return pl.pallas_call(
        body,
        grid=(n // _BN,),
        in_specs=[pl.BlockSpec((_BN, d), lambda i: (i, 0)),
                  pl.BlockSpec((_BN, d), lambda i: (i, 0)),
                  pl.BlockSpec((_BN, 1), lambda i: (i, 0)),
                  pl.BlockSpec((_BN, d), lambda i: (i, 0)),
                  pl.BlockSpec((1, d), lambda i: (0, 0))],
        out_specs=pl.BlockSpec((_BN, d), lambda i: (i, 0)),
        out_shape=jax.ShapeDtypeStruct((n, d), jnp.float32),
    )(acc_a, acc_b, dis, h2, b2)


def kernel(x, edge_index, W1, b1, W2, b2):
    n, _ = x.shape
    e = edge_index.shape[1]
    src = edge_index[0].astype(jnp.int32).reshape(e // K, K)
    dst = edge_index[1].astype(jnp.int32).reshape(e // K, K)

    zeros16 = jnp.zeros((n, 16), jnp.float32)
    onesk = jnp.ones((K, 16), jnp.float32)
    zeros_h = jnp.zeros((n, W1.shape[1]), jnp.float32)
    zeros_o = jnp.zeros((n, W2.shape[1]), jnp.float32)

    degacc = _sc_degree(dst, onesk, zeros16, n)
    h1 = _tc_matmul(x, W1)
    dis, hp1 = _tc_prep(degacc[0], degacc[1], h1)
    acc1 = _sc_propagate(hp1, src, dst, zeros_h, n, W1.shape[1])
    h2, hp2 = _tc_mid(acc1[0], acc1[1], dis, h1, b1.reshape(1, -1), W2)
    acc2 = _sc_propagate(hp2, src, dst, zeros_o, n, W2.shape[1])
    return _tc_final(acc2[0], acc2[1], dis, h2, b2.reshape(1, -1))


# speedup vs baseline: 12.0738x; 12.0738x over previous
import functools

import jax
import jax.numpy as jnp
from jax import lax
from jax.experimental import pallas as pl
from jax.experimental.pallas import tpu as pltpu
from jax.experimental.pallas import tpu_sc as plsc

NC = 2
NS = 16
NW = NC * NS
K = 80


def _sc_mesh():
    return plsc.VectorSubcoreMesh(core_axis_name="c", subcore_axis_name="s")


def _sc_degree(dst1d, ones, zeros, npad):
    e = dst1d.shape[0]
    epw = e // NW
    bpw = epw // K
    rps = npad // NS

    @functools.partial(
        pl.kernel,
        out_type=jax.ShapeDtypeStruct((NC, npad, 128), jnp.float32),
        mesh=_sc_mesh(),
        scratch_types=[
            pltpu.VMEM((K,), jnp.int32),
            pltpu.VMEM((K, 128), jnp.float32),
            pltpu.VMEM_SHARED((npad, 128), jnp.float32),
            pltpu.SemaphoreType.DMA,
        ],
    )
    def k(dst_hbm, ones_hbm, zeros_hbm, out_hbm, dstb, ones_v, acc_sh, sem):
        c = lax.axis_index("c")
        s = lax.axis_index("s")
        w = c * NS + s
        pltpu.sync_copy(ones_hbm, ones_v)
        pltpu.sync_copy(zeros_hbm.at[pl.ds(s * rps, rps)],
                        acc_sh.at[pl.ds(s * rps, rps)])
        plsc.subcore_barrier()

        @pl.loop(0, bpw)
        def _(j):
            pltpu.sync_copy(dst_hbm.at[pl.ds(w * epw + j * K, K)], dstb)
            pltpu.sync_copy(ones_v, acc_sh.at[dstb], add=True)

        plsc.subcore_barrier()
        pltpu.sync_copy(acc_sh.at[pl.ds(s * rps, rps)],
                        out_hbm.at[c, pl.ds(s * rps, rps)])

    return k(dst1d, ones, zeros)


def _sc_propagate(hp, src1d, dst1d, zeros, npad, d):
    e = src1d.shape[0]
    epw = e // NW
    bpw = epw // K
    rps = npad // NS

    @functools.partial(
        pl.kernel,
        out_type=jax.ShapeDtypeStruct((NC, npad, d), jnp.float32),
        mesh=_sc_mesh(),
        scratch_types=[
            pltpu.VMEM((K,), jnp.int32),
            pltpu.VMEM((K,), jnp.int32),
            pltpu.VMEM((K, d), jnp.float32),
            pltpu.VMEM_SHARED((npad, d), jnp.float32),
            pltpu.SemaphoreType.DMA,
        ],
    )
    def k(hp_hbm, src_hbm, dst_hbm, zeros_hbm, out_hbm,
          srcb, dstb, rows_v, acc_sh, sem):
        c = lax.axis_index("c")
        s = lax.axis_index("s")
        w = c * NS + s
        pltpu.sync_copy(zeros_hbm.at[pl.ds(s * rps, rps)],
                        acc_sh.at[pl.ds(s * rps, rps)])
        plsc.subcore_barrier()

        @pl.loop(0, bpw)
        def _(j):
            pltpu.sync_copy(src_hbm.at[pl.ds(w * epw + j * K, K)], srcb)
            pltpu.sync_copy(dst_hbm.at[pl.ds(w * epw + j * K, K)], dstb)
            pltpu.async_copy(hp_hbm.at[srcb], rows_v, sem).wait()
            pltpu.sync_copy(rows_v, acc_sh.at[dstb], add=True)

        plsc.subcore_barrier()
        pltpu.sync_copy(acc_sh.at[pl.ds(s * rps, rps)],
                        out_hbm.at[c, pl.ds(s * rps, rps)])

    return k(hp, src1d, dst1d, zeros)


_BN = 1000


def _tc_matmul(x, w):
    n, kd = x.shape
    dout = w.shape[1]

    def body(x_ref, w_ref, o_ref):
        o_ref[...] = jnp.dot(x_ref[...], w_ref[...],
                             preferred_element_type=jnp.float32,
                             precision=lax.Precision.HIGHEST)

    return pl.pallas_call(
        body,
        grid=(n // _BN,),
        in_specs=[pl.BlockSpec((_BN, kd), lambda i: (i, 0)),
                  pl.BlockSpec((kd, dout), lambda i: (0, 0))],
        out_specs=pl.BlockSpec((_BN, dout), lambda i: (i, 0)),
        out_shape=jax.ShapeDtypeStruct((n, dout), jnp.float32),
    )(x, w)


def _tc_prep(deg0, deg1, h1):
    n, d = h1.shape

    def body(d0_ref, d1_ref, h_ref, dis_ref, hp_ref):
        deg = d0_ref[:, :1] + d1_ref[:, :1] + 1.0
        dis = lax.rsqrt(deg)
        dis_ref[...] = dis
        hp_ref[...] = dis * h_ref[...]

    return pl.pallas_call(
        body,
        grid=(n // _BN,),
        in_specs=[pl.BlockSpec((_BN, 16), lambda i: (i, 0)),
                  pl.BlockSpec((_BN, 16), lambda i: (i, 0)),
                  pl.BlockSpec((_BN, d), lambda i: (i, 0))],
        out_specs=[pl.BlockSpec((_BN, 1), lambda i: (i, 0)),
                   pl.BlockSpec((_BN, d), lambda i: (i, 0))],
        out_shape=[jax.ShapeDtypeStruct((n, 1), jnp.float32),
                   jax.ShapeDtypeStruct((n, d), jnp.float32)],
    )(deg0, deg1, h1)


def _tc_mid(acc_a, acc_b, dis, h1, b1, w2):
    n, d = h1.shape
    dout = w2.shape[1]

    def body(a_ref, b_ref, dis_ref, h1_ref, bias_ref, w2_ref, h2_ref, hp2_ref):
        dis = dis_ref[...]
        z = dis * (a_ref[...] + b_ref[...]) + (dis * dis) * h1_ref[...] \
            + bias_ref[...]
        h = jnp.maximum(z, 0.0)
        h2 = jnp.dot(h, w2_ref[...], preferred_element_type=jnp.float32,
                     precision=lax.Precision.HIGHEST)
        h2_ref[...] = h2
        hp2_ref[...] = dis * h2

    return pl.pallas_call(
        body,
        grid=(n // _BN,),
        in_specs=[pl.BlockSpec((_BN, d), lambda i: (i, 0)),
                  pl.BlockSpec((_BN, d), lambda i: (i, 0)),
                  pl.BlockSpec((_BN, 1), lambda i: (i, 0)),
                  pl.BlockSpec((_BN, d), lambda i: (i, 0)),
                  pl.BlockSpec((1, d), lambda i: (0, 0)),
                  pl.BlockSpec((d, dout), lambda i: (0, 0))],
        out_specs=[pl.BlockSpec((_BN, dout), lambda i: (i, 0)),
                   pl.BlockSpec((_BN, dout), lambda i: (i, 0))],
        out_shape=[jax.ShapeDtypeStruct((n, dout), jnp.float32),
                   jax.ShapeDtypeStruct((n, dout), jnp.float32)],
    )(acc_a, acc_b, dis, h1, b1, w2)


def _tc_final(acc_a, acc_b, dis, h2, b2):
    n, d = h2.shape

    def body(a_ref, b_ref, dis_ref, h2_ref, bias_ref, o_ref):
        dis = dis_ref[...]
        z = dis * (a_ref[...] + b_ref[...]) + (dis * dis) * h2_ref[...] \
            + bias_ref[...]
        m = jnp.max(z, axis=1, keepdims=True)
        lse = jnp.log(jnp.sum(jnp.exp(z - m), axis=1, keepdims=True)) + m
        o_ref[...] = z - lse

    return pl.pallas_call(
        body,
        grid=(n // _BN,),
        in_specs=[pl.BlockSpec((_BN, d), lambda i: (i, 0)),
                  pl.BlockSpec((_BN, d), lambda i: (i, 0)),
                  pl.BlockSpec((_BN, 1), lambda i: (i, 0)),
                  pl.BlockSpec((_BN, d), lambda i: (i, 0)),
                  pl.BlockSpec((1, d), lambda i: (0, 0))],
        out_specs=pl.BlockSpec((_BN, d), lambda i: (i, 0)),
        out_shape=jax.ShapeDtypeStruct((n, d), jnp.float32),
    )(acc_a, acc_b, dis, h2, b2)


def kernel(x, edge_index, W1, b1, W2, b2):
    n, _ = x.shape
    npad = ((n + 8 * NS - 1) // (8 * NS)) * (8 * NS)
    src = edge_index[0].astype(jnp.int32)
    dst = edge_index[1].astype(jnp.int32)

    zeros16 = jnp.zeros((npad, 128), jnp.float32)
    onesk = jnp.ones((K, 128), jnp.float32)
    zeros_h = jnp.zeros((npad, W1.shape[1]), jnp.float32)
    zeros_o = jnp.zeros((npad, W2.shape[1]), jnp.float32)

    degacc = _sc_degree(dst, onesk, zeros16, npad)
    h1 = _tc_matmul(x, W1)
    dis, hp1 = _tc_prep(degacc[0, :n, :16], degacc[1, :n, :16], h1)
    acc1 = _sc_propagate(hp1, src, dst, zeros_h, npad, W1.shape[1])
    h2, hp2 = _tc_mid(acc1[0, :n], acc1[1, :n], dis, h1,
                      b1.reshape(1, -1), W2)
    dpad = 128
    hp2p = jnp.pad(hp2, ((0, 0), (0, dpad - hp2.shape[1])))
    acc2 = _sc_propagate(hp2p, src, dst, zeros_h, npad, dpad)
    return _tc_final(acc2[0, :n, :W2.shape[1]], acc2[1, :n, :W2.shape[1]],
                     dis, h2, b2.reshape(1, -1))


# speedup vs baseline: 24.3938x; 2.0204x over previous
import functools

import jax
import jax.numpy as jnp
from jax import lax
from jax.experimental import pallas as pl
from jax.experimental.pallas import tpu as pltpu
from jax.experimental.pallas import tpu_sc as plsc

NC = 2
NS = 16
NW = NC * NS
K = 80


def _sc_mesh():
    return plsc.VectorSubcoreMesh(core_axis_name="c", subcore_axis_name="s")


def _sc_degree(dst3d, ones, zeros, npad):
    bpw = dst3d.shape[1]
    rps = npad // NS

    @functools.partial(
        pl.kernel,
        out_type=jax.ShapeDtypeStruct((NC, npad, 128), jnp.float32),
        mesh=_sc_mesh(),
        scratch_types=[
            pltpu.VMEM((bpw, K), jnp.int32),
            pltpu.VMEM((K, 128), jnp.float32),
            pltpu.VMEM_SHARED((npad, 128), jnp.float32),
            pltpu.SemaphoreType.DMA,
        ],
    )
    def k(dst_hbm, ones_hbm, zeros_hbm, out_hbm, dst_v, ones_v, acc_sh, sem):
        c = lax.axis_index("c")
        s = lax.axis_index("s")
        w = c * NS + s
        pltpu.sync_copy(dst_hbm.at[w], dst_v)
        pltpu.sync_copy(ones_hbm, ones_v)
        pltpu.sync_copy(zeros_hbm.at[pl.ds(s * rps, rps)],
                        acc_sh.at[pl.ds(s * rps, rps)])
        plsc.subcore_barrier()

        @pl.loop(0, bpw)
        def _(j):
            pltpu.sync_copy(ones_v, acc_sh.at[dst_v.at[j]], add=True)

        plsc.subcore_barrier()
        pltpu.sync_copy(acc_sh.at[pl.ds(s * rps, rps)],
                        out_hbm.at[c, pl.ds(s * rps, rps)])

    return k(dst3d, ones, zeros)


def _sc_propagate(hp, src1d, dst3d, zeros, npad, d):
    e = src1d.shape[0]
    epw = e // NW
    bpw = dst3d.shape[1]
    rps = npad // NS
    nbuf = 2
    nout = (bpw + nbuf - 1) // nbuf

    row_t = [pltpu.VMEM((K, d), jnp.float32) for _ in range(nbuf)]
    sem_t = [pltpu.SemaphoreType.DMA for _ in range(nbuf)]

    @functools.partial(
        pl.kernel,
        out_type=jax.ShapeDtypeStruct((NC, npad, d), jnp.float32),
        mesh=_sc_mesh(),
        scratch_types=[
            pltpu.VMEM((epw,), jnp.int32),
            pltpu.VMEM((bpw, K), jnp.int32),
            pltpu.VMEM_SHARED((npad, d), jnp.float32),
        ] + row_t + sem_t,
    )
    def k(hp_hbm, src_hbm, dst_hbm, zeros_hbm, out_hbm,
          src_v, dst_v, acc_sh, *bufs_and_sems):
        rows = bufs_and_sems[:nbuf]
        gs = bufs_and_sems[nbuf:]
        c = lax.axis_index("c")
        s = lax.axis_index("s")
        w = c * NS + s
        pltpu.sync_copy(src_hbm.at[pl.ds(w * epw, epw)], src_v)
        pltpu.sync_copy(dst_hbm.at[w], dst_v)
        pltpu.sync_copy(zeros_hbm.at[pl.ds(s * rps, rps)],
                        acc_sh.at[pl.ds(s * rps, rps)])
        plsc.subcore_barrier()

        def gather(j, b):
            return pltpu.async_copy(
                hp_hbm.at[src_v.at[pl.ds(j * K, K)]], rows[b], gs[b])

        for b in range(nbuf):
            gather(b, b)

        @pl.loop(0, nout)
        def _(i):
            for b in range(nbuf):
                j = i * nbuf + b

                @pl.when(j < bpw)
                def _():
                    pltpu.make_async_copy(
                        hp_hbm.at[src_v.at[pl.ds(j * K, K)]],
                        rows[b], gs[b]).wait()
                    pltpu.sync_copy(rows[b], acc_sh.at[dst_v.at[j]],
                                    add=True)

                    @pl.when(j + nbuf < bpw)
                    def _():
                        gather(j + nbuf, b)

        plsc.subcore_barrier()
        pltpu.sync_copy(acc_sh.at[pl.ds(s * rps, rps)],
                        out_hbm.at[c, pl.ds(s * rps, rps)])

    return k(hp, src1d, dst3d, zeros)


_BN = 1000


def _tc_matmul(x, w):
    n, kd = x.shape
    dout = w.shape[1]

    def body(x_ref, w_ref, o_ref):
        o_ref[...] = jnp.dot(x_ref[...], w_ref[...],
                             preferred_element_type=jnp.float32,
                             precision=lax.Precision.HIGHEST)

    return pl.pallas_call(
        body,
        grid=(n // _BN,),
        in_specs=[pl.BlockSpec((_BN, kd), lambda i: (i, 0)),
                  pl.BlockSpec((kd, dout), lambda i: (0, 0))],
        out_specs=pl.BlockSpec((_BN, dout), lambda i: (i, 0)),
        out_shape=jax.ShapeDtypeStruct((n, dout), jnp.float32),
    )(x, w)


def _tc_prep(deg0, deg1, h1):
    n, d = h1.shape

    def body(d0_ref, d1_ref, h_ref, dis_ref, hp_ref):
        deg = d0_ref[:, :1] + d1_ref[:, :1] + 1.0
        dis = lax.rsqrt(deg)
        dis_ref[...] = dis
        hp_ref[...] = dis * h_ref[...]

    return pl.pallas_call(
        body,
        grid=(n // _BN,),
        in_specs=[pl.BlockSpec((_BN, 16), lambda i: (i, 0)),
                  pl.BlockSpec((_BN, 16), lambda i: (i, 0)),
                  pl.BlockSpec((_BN, d), lambda i: (i, 0))],
        out_specs=[pl.BlockSpec((_BN, 1), lambda i: (i, 0)),
                   pl.BlockSpec((_BN, d), lambda i: (i, 0))],
        out_shape=[jax.ShapeDtypeStruct((n, 1), jnp.float32),
                   jax.ShapeDtypeStruct((n, d), jnp.float32)],
    )(deg0, deg1, h1)


def _tc_mid(acc_a, acc_b, dis, h1, b1, w2):
    n, d = h1.shape
    dout = w2.shape[1]

    def body(a_ref, b_ref, dis_ref, h1_ref, bias_ref, w2_ref, h2_ref, hp2_ref):
        dis = dis_ref[...]
        z = dis * (a_ref[...] + b_ref[...]) + (dis * dis) * h1_ref[...] \
            + bias_ref[...]
        h = jnp.maximum(z, 0.0)
        h2 = jnp.dot(h, w2_ref[...], preferred_element_type=jnp.float32,
                     precision=lax.Precision.HIGHEST)
        h2_ref[...] = h2
        hp2_ref[...] = dis * h2

    return pl.pallas_call(
        body,
        grid=(n // _BN,),
        in_specs=[pl.BlockSpec((_BN, d), lambda i: (i, 0)),
                  pl.BlockSpec((_BN, d), lambda i: (i, 0)),
                  pl.BlockSpec((_BN, 1), lambda i: (i, 0)),
                  pl.BlockSpec((_BN, d), lambda i: (i, 0)),
                  pl.BlockSpec((1, d), lambda i: (0, 0)),
                  pl.BlockSpec((d, dout), lambda i: (0, 0))],
        out_specs=[pl.BlockSpec((_BN, dout), lambda i: (i, 0)),
                   pl.BlockSpec((_BN, dout), lambda i: (i, 0))],
        out_shape=[jax.ShapeDtypeStruct((n, dout), jnp.float32),
                   jax.ShapeDtypeStruct((n, dout), jnp.float32)],
    )(acc_a, acc_b, dis, h1, b1, w2)


def _tc_final(acc_a, acc_b, dis, h2, b2):
    n, d = h2.shape

    def body(a_ref, b_ref, dis_ref, h2_ref, bias_ref, o_ref):
        dis = dis_ref[...]
        z = dis * (a_ref[...] + b_ref[...]) + (dis * dis) * h2_ref[...] \
            + bias_ref[...]
        m = jnp.max(z, axis=1, keepdims=True)
        lse = jnp.log(jnp.sum(jnp.exp(z - m), axis=1, keepdims=True)) + m
        o_ref[...] = z - lse

    return pl.pallas_call(
        body,
        grid=(n // _BN,),
        in_specs=[pl.BlockSpec((_BN, d), lambda i: (i, 0)),
                  pl.BlockSpec((_BN, d), lambda i: (i, 0)),
                  pl.BlockSpec((_BN, 1), lambda i: (i, 0)),
                  pl.BlockSpec((_BN, d), lambda i: (i, 0)),
                  pl.BlockSpec((1, d), lambda i: (0, 0))],
        out_specs=pl.BlockSpec((_BN, d), lambda i: (i, 0)),
        out_shape=jax.ShapeDtypeStruct((n, d), jnp.float32),
    )(acc_a, acc_b, dis, h2, b2)


def kernel(x, edge_index, W1, b1, W2, b2):
    n, _ = x.shape
    npad = ((n + 8 * NS - 1) // (8 * NS)) * (8 * NS)
    e = edge_index.shape[1]
    src = edge_index[0].astype(jnp.int32)
    dst3 = edge_index[1].astype(jnp.int32).reshape(NW, e // (NW * K), K)

    zeros16 = jnp.zeros((npad, 128), jnp.float32)
    onesk = jnp.ones((K, 128), jnp.float32)
    zeros_h = jnp.zeros((npad, W1.shape[1]), jnp.float32)
    zeros_o = jnp.zeros((npad, W2.shape[1]), jnp.float32)

    degacc = _sc_degree(dst3, onesk, zeros16, npad)
    h1 = _tc_matmul(x, W1)
    dis, hp1 = _tc_prep(degacc[0, :n, :16], degacc[1, :n, :16], h1)
    acc1 = _sc_propagate(hp1, src, dst3, zeros_h, npad, W1.shape[1])
    h2, hp2 = _tc_mid(acc1[0, :n], acc1[1, :n], dis, h1,
                      b1.reshape(1, -1), W2)
    dpad = 128
    hp2p = jnp.pad(hp2, ((0, 0), (0, dpad - hp2.shape[1])))
    acc2 = _sc_propagate(hp2p, src, dst3, zeros_h, npad, dpad)
    return _tc_final(acc2[0, :n, :W2.shape[1]], acc2[1, :n, :W2.shape[1]],
                     dis, h2, b2.reshape(1, -1))


# speedup vs baseline: 25.8085x; 1.0580x over previous
import functools

import jax
import jax.numpy as jnp
from jax import lax
from jax.experimental import pallas as pl
from jax.experimental.pallas import tpu as pltpu
from jax.experimental.pallas import tpu_sc as plsc

NC = 2
NS = 16
NW = NC * NS
K = 80


def _sc_mesh():
    return plsc.VectorSubcoreMesh(core_axis_name="c", subcore_axis_name="s")


def _sc_degree(dst3d, ones, zeros, npad):
    bpw = dst3d.shape[1]
    rps = npad // NS

    @functools.partial(
        pl.kernel,
        out_type=jax.ShapeDtypeStruct((NC, npad, 128), jnp.float32),
        mesh=_sc_mesh(),
        scratch_types=[
            pltpu.VMEM((bpw, K), jnp.int32),
            pltpu.VMEM((K, 128), jnp.float32),
            pltpu.VMEM_SHARED((npad, 128), jnp.float32),
            pltpu.SemaphoreType.DMA,
        ],
    )
    def k(dst_hbm, ones_hbm, zeros_hbm, out_hbm, dst_v, ones_v, acc_sh, sem):
        c = lax.axis_index("c")
        s = lax.axis_index("s")
        w = c * NS + s
        pltpu.sync_copy(dst_hbm.at[w], dst_v)
        pltpu.sync_copy(ones_hbm, ones_v)
        pltpu.sync_copy(zeros_hbm.at[pl.ds(s * rps, rps)],
                        acc_sh.at[pl.ds(s * rps, rps)])
        plsc.subcore_barrier()

        @pl.loop(0, bpw)
        def _(j):
            pltpu.sync_copy(ones_v, acc_sh.at[dst_v.at[j]], add=True)

        plsc.subcore_barrier()
        pltpu.sync_copy(acc_sh.at[pl.ds(s * rps, rps)],
                        out_hbm.at[c, pl.ds(s * rps, rps)])

    return k(dst3d, ones, zeros)


def _sc_propagate(hp, src1d, dst3d, zeros, npad, d):
    e = src1d.shape[0]
    epw = e // NW
    bpw = dst3d.shape[1]
    rps = npad // NS
    nbuf = 2
    nout = (bpw + nbuf - 1) // nbuf

    row_t = [pltpu.VMEM((K, d), jnp.float32) for _ in range(nbuf)]
    sem_t = [pltpu.SemaphoreType.DMA for _ in range(nbuf)]

    @functools.partial(
        pl.kernel,
        out_type=jax.ShapeDtypeStruct((NC, npad, d), jnp.float32),
        mesh=_sc_mesh(),
        scratch_types=[
            pltpu.VMEM((epw,), jnp.int32),
            pltpu.VMEM((bpw, K), jnp.int32),
            pltpu.VMEM_SHARED((npad, d), jnp.float32),
        ] + row_t + sem_t,
    )
    def k(hp_hbm, src_hbm, dst_hbm, zeros_hbm, out_hbm,
          src_v, dst_v, acc_sh, *bufs_and_sems):
        rows = bufs_and_sems[:nbuf]
        gs = bufs_and_sems[nbuf:]
        c = lax.axis_index("c")
        s = lax.axis_index("s")
        w = c * NS + s
        pltpu.sync_copy(src_hbm.at[pl.ds(w * epw, epw)], src_v)
        pltpu.sync_copy(dst_hbm.at[w], dst_v)
        pltpu.sync_copy(zeros_hbm.at[pl.ds(s * rps, rps)],
                        acc_sh.at[pl.ds(s * rps, rps)])
        plsc.subcore_barrier()

        def gather(j, b):
            return pltpu.async_copy(
                hp_hbm.at[src_v.at[pl.ds(j * K, K)]], rows[b], gs[b])

        for b in range(nbuf):
            gather(b, b)

        @pl.loop(0, nout)
        def _(i):
            for b in range(nbuf):
                j = i * nbuf + b

                @pl.when(j < bpw)
                def _():
                    pltpu.make_async_copy(
                        hp_hbm.at[src_v.at[pl.ds(j * K, K)]],
                        rows[b], gs[b]).wait()
                    pltpu.sync_copy(rows[b], acc_sh.at[dst_v.at[j]],
                                    add=True)

                    @pl.when(j + nbuf < bpw)
                    def _():
                        gather(j + nbuf, b)

        plsc.subcore_barrier()
        pltpu.sync_copy(acc_sh.at[pl.ds(s * rps, rps)],
                        out_hbm.at[c, pl.ds(s * rps, rps)])

    return k(hp, src1d, dst3d, zeros)


_BN = 1000


def _tc_prep(x, w1, degacc):
    n, kd = x.shape
    d = w1.shape[1]

    def body(x_ref, w_ref, d0_ref, d1_ref, dis_ref, h1_ref, hp_ref):
        h1 = jnp.dot(x_ref[...], w_ref[...],
                     preferred_element_type=jnp.float32,
                     precision=lax.Precision.HIGHEST)
        deg = d0_ref[0, :, :1] + d1_ref[0, :, :1] + 1.0
        dis = lax.rsqrt(deg)
        dis_ref[...] = dis
        h1_ref[...] = h1
        hp_ref[...] = dis * h1

    return pl.pallas_call(
        body,
        grid=(n // _BN,),
        in_specs=[pl.BlockSpec((_BN, kd), lambda i: (i, 0)),
                  pl.BlockSpec((kd, d), lambda i: (0, 0)),
                  pl.BlockSpec((1, _BN, 128), lambda i: (0, i, 0)),
                  pl.BlockSpec((1, _BN, 128), lambda i: (1, i, 0))],
        out_specs=[pl.BlockSpec((_BN, 1), lambda i: (i, 0)),
                   pl.BlockSpec((_BN, d), lambda i: (i, 0)),
                   pl.BlockSpec((_BN, d), lambda i: (i, 0))],
        out_shape=[jax.ShapeDtypeStruct((n, 1), jnp.float32),
                   jax.ShapeDtypeStruct((n, d), jnp.float32),
                   jax.ShapeDtypeStruct((n, d), jnp.float32)],
    )(x, w1, degacc, degacc)


def _tc_mid(acc1, dis, h1, b1, w2):
    n, d = h1.shape
    dout = w2.shape[1]

    def body(a_ref, b_ref, dis_ref, h1_ref, bias_ref, w2_ref, h2_ref, hp2_ref):
        dis = dis_ref[...]
        z = dis * (a_ref[0] + b_ref[0]) + (dis * dis) * h1_ref[...] \
            + bias_ref[...]
        h = jnp.maximum(z, 0.0)
        h2 = jnp.dot(h, w2_ref[...], preferred_element_type=jnp.float32,
                     precision=lax.Precision.HIGHEST)
        h2_ref[...] = h2
        hp2 = dis * h2
        hp2_ref[...] = jnp.concatenate(
            [hp2, jnp.zeros((hp2.shape[0], d - dout), jnp.float32)], axis=1)

    return pl.pallas_call(
        body,
        grid=(n // _BN,),
        in_specs=[pl.BlockSpec((1, _BN, d), lambda i: (0, i, 0)),
                  pl.BlockSpec((1, _BN, d), lambda i: (1, i, 0)),
                  pl.BlockSpec((_BN, 1), lambda i: (i, 0)),
                  pl.BlockSpec((_BN, d), lambda i: (i, 0)),
                  pl.BlockSpec((1, d), lambda i: (0, 0)),
                  pl.BlockSpec((d, dout), lambda i: (0, 0))],
        out_specs=[pl.BlockSpec((_BN, dout), lambda i: (i, 0)),
                   pl.BlockSpec((_BN, d), lambda i: (i, 0))],
        out_shape=[jax.ShapeDtypeStruct((n, dout), jnp.float32),
                   jax.ShapeDtypeStruct((n, d), jnp.float32)],
    )(acc1, acc1, dis, h1, b1, w2)


def _tc_final(acc2, dis, h2, b2):
    n, d = h2.shape

    def body(a_ref, b_ref, dis_ref, h2_ref, bias_ref, o_ref):
        dis = dis_ref[...]
        z = dis * (a_ref[0, :, :h2_ref.shape[1]] +
                   b_ref[0, :, :h2_ref.shape[1]]) \
            + (dis * dis) * h2_ref[...] + bias_ref[...]
        m = jnp.max(z, axis=1, keepdims=True)
        lse = jnp.log(jnp.sum(jnp.exp(z - m), axis=1, keepdims=True)) + m
        o_ref[...] = z - lse

    return pl.pallas_call(
        body,
        grid=(n // _BN,),
        in_specs=[pl.BlockSpec((1, _BN, 128), lambda i: (0, i, 0)),
                  pl.BlockSpec((1, _BN, 128), lambda i: (1, i, 0)),
                  pl.BlockSpec((_BN, 1), lambda i: (i, 0)),
                  pl.BlockSpec((_BN, d), lambda i: (i, 0)),
                  pl.BlockSpec((1, d), lambda i: (0, 0))],
        out_specs=pl.BlockSpec((_BN, d), lambda i: (i, 0)),
        out_shape=jax.ShapeDtypeStruct((n, d), jnp.float32),
    )(acc2, acc2, dis, h2, b2)


def kernel(x, edge_index, W1, b1, W2, b2):
    n, _ = x.shape
    npad = ((n + 8 * NS - 1) // (8 * NS)) * (8 * NS)
    e = edge_index.shape[1]
    src = edge_index[0].astype(jnp.int32)
    dst3 = edge_index[1].astype(jnp.int32).reshape(NW, e // (NW * K), K)

    onesk = jnp.ones((K, 128), jnp.float32)
    zeros_h = jnp.zeros((npad, 128), jnp.float32)

    degacc = _sc_degree(dst3, onesk, zeros_h, npad)
    dis, h1, hp1 = _tc_prep(x, W1, degacc)
    acc1 = _sc_propagate(hp1, src, dst3, zeros_h, npad, 128)
    h2, hp2p = _tc_mid(acc1, dis, h1, b1.reshape(1, -1), W2)
    acc2 = _sc_propagate(hp2p, src, dst3, zeros_h, npad, 128)
    return _tc_final(acc2, dis, h2, b2.reshape(1, -1))
